# Initial kernel scaffold; baseline (speedup 1.0000x reference)
import jax
import jax.numpy as jnp
from jax.experimental import pallas as pl


def kernel(cls_0, cls_1, cls_2, cls_3, cls_4, box_0, box_1, box_2, box_3, box_4):
    raise NotImplementedError("write your pallas kernel here")


# speedup vs baseline: 1.1607x; 1.1607x over previous
import jax
import jax.numpy as jnp
from jax.experimental import pallas as pl
from jax.experimental.pallas import tpu as pltpu

NC = 90
K = 5000
INT_MIN = -2147483648
INT_MAX = 2147483647
PAD_KEY = -2139095041
SLAB = 332


def _select_kernel(key_ref, vals_ref, fidx_ref, itie_ref):
    nchunks = key_ref.shape[1]
    n_pad = nchunks * 1024
    nslabs = nchunks // SLAB

    def count_pred(pred_fn):
        def sbody(s, acc):
            sl = key_ref[0, pl.ds(s * SLAB, SLAB)]
            return acc + jnp.sum(pred_fn(s, sl).astype(jnp.int32))
        return jax.lax.fori_loop(0, nslabs, sbody, jnp.int32(0))

    def count_ge(t):
        return count_pred(lambda s, sl: sl >= t)

    c_pos = count_ge(jnp.int32(0))
    pos_branch = c_pos >= K
    lo0 = jnp.where(pos_branch, jnp.int32(0), jnp.int32(INT_MIN))
    hi0 = jnp.where(pos_branch, jnp.int32(INT_MAX), jnp.int32(-1))

    def bbody(_, lh):
        lo, hi = lh
        d = hi - lo
        mid = lo + (d >> 1) + (d & 1)
        pred = count_ge(mid) >= K
        return (jnp.where(pred, mid, lo), jnp.where(pred, hi, mid - 1))

    t_val, _ = jax.lax.fori_loop(0, 31, bbody, (lo0, hi0))

    c_gt = count_pred(lambda s, sl: sl > t_val)
    c_ge = count_pred(lambda s, sl: sl >= t_val)
    need = K - c_gt

    itie_ref[0] = jnp.int32(n_pad)

    @pl.when(c_ge > K)
    def _tie_search():
        def tbody(_, lh):
            lo, hi = lh
            mid = (lo + hi) >> 1

            def pred_fn(s, sl):
                g = (s * (SLAB * 1024)
                     + jax.lax.broadcasted_iota(jnp.int32, sl.shape, 0) * 1024
                     + jax.lax.broadcasted_iota(jnp.int32, sl.shape, 1) * 128
                     + jax.lax.broadcasted_iota(jnp.int32, sl.shape, 2))
                return (sl == t_val) & (g < mid)

            pred = count_pred(pred_fn) >= need
            return (jnp.where(pred, lo, mid + 1), jnp.where(pred, mid, hi))

        tlo, _ = jax.lax.fori_loop(0, 23, tbody,
                                   (jnp.int32(0), jnp.int32(n_pad)))
        itie_ref[0] = tlo

    i_tie = itie_ref[0]

    liota = (jax.lax.broadcasted_iota(jnp.int32, (8, 128), 0) * 128
             + jax.lax.broadcasted_iota(jnp.int32, (8, 128), 1))

    def chunk_body(r, wpos):
        chunk = key_ref[0, r]
        gidx = r * 1024 + liota
        sel = (chunk > t_val) | ((chunk == t_val) & (gidx < i_tie))
        cnt = jnp.sum(sel.astype(jnp.int32))
        xc = jax.lax.bitcast_convert_type(
            chunk ^ ((chunk >> 31) & jnp.int32(0x7FFFFFFF)), jnp.float32)
        work0 = jnp.where(sel, chunk, jnp.int32(INT_MIN))

        def extract(_, carry):
            work, wp = carry
            mx = jnp.max(work)
            pos = jnp.min(jnp.where(work == mx, liota, jnp.int32(1024)))
            val = jnp.sum(jnp.where(liota == pos, xc, 0.0))
            vals_ref[0, pl.ds(wp, 1), :] = jnp.full((1, 128), val, jnp.float32)
            fidx_ref[0, pl.ds(wp, 1), :] = jnp.full((1, 128), r * 1024 + pos,
                                                    jnp.int32)
            work = jnp.where(liota == pos, jnp.int32(INT_MIN), work)
            return (work, wp + 1)

        _, wpos = jax.lax.fori_loop(0, cnt, extract, (work0, wpos))
        return wpos

    jax.lax.fori_loop(0, nchunks, chunk_body, jnp.int32(0))


def kernel(cls_0, cls_1, cls_2, cls_3, cls_4, box_0, box_1, box_2, box_3, box_4):
    cls_outputs = [cls_0, cls_1, cls_2, cls_3, cls_4]
    box_outputs = [box_0, box_1, box_2, box_3, box_4]
    batch = cls_0.shape[0]
    cls_all = jnp.concatenate(
        [jnp.transpose(c, (0, 2, 3, 1)).reshape(batch, -1, NC) for c in cls_outputs],
        axis=1)
    box_all = jnp.concatenate(
        [jnp.transpose(b, (0, 2, 3, 1)).reshape(batch, -1, 4) for b in box_outputs],
        axis=1)
    n = cls_all.shape[1] * NC
    nchunks = (n + 1023) // 1024
    if nchunks % SLAB:
        nchunks += SLAB - nchunks % SLAB
    n_pad = nchunks * 1024
    cls_flat = cls_all.reshape(batch, n)
    xi = jax.lax.bitcast_convert_type(cls_flat, jnp.int32)
    skey = xi ^ ((xi >> 31) & jnp.int32(0x7FFFFFFF))
    skey = jnp.pad(skey, ((0, 0), (0, n_pad - n)), constant_values=PAD_KEY)
    skey_blocked = skey.reshape(batch, nchunks, 8, 128)

    vals, fidx = pl.pallas_call(
        _select_kernel,
        grid=(batch,),
        in_specs=[pl.BlockSpec((1, nchunks, 8, 128), lambda b: (b, 0, 0, 0))],
        out_specs=[
            pl.BlockSpec((1, K, 128), lambda b: (b, 0, 0)),
            pl.BlockSpec((1, K, 128), lambda b: (b, 0, 0)),
        ],
        out_shape=[
            jax.ShapeDtypeStruct((batch, K, 128), jnp.float32),
            jax.ShapeDtypeStruct((batch, K, 128), jnp.int32),
        ],
        scratch_shapes=[
            pltpu.SMEM((1,), jnp.int32),
        ],
        compiler_params=pltpu.CompilerParams(vmem_limit_bytes=60 * 1024 * 1024),
    )(skey_blocked)
    vals = vals[:, :, 0]
    fidx = fidx[:, :, 0]

    ord_vals, ord_pos = jax.lax.top_k(vals, K)
    indices_flat = jnp.take_along_axis(fidx, ord_pos, axis=1)
    anchors = indices_flat // NC
    classes = indices_flat % NC
    box_topk = jnp.take_along_axis(box_all, anchors[..., None], axis=1)
    cls_topk = ord_vals[..., None]
    return (cls_topk, box_topk, anchors, classes)


# speedup vs baseline: 1.1609x; 1.0002x over previous
import jax
import jax.numpy as jnp
from jax.experimental import pallas as pl
from jax.experimental.pallas import tpu as pltpu

NC = 90
K = 5000
INT_MIN = -2147483648
INT_MAX = 2147483647
PAD_KEY = -2139095041
SLAB = 332


def _select_kernel(key_ref, vals_ref, fidx_ref, itie_ref):
    nchunks = key_ref.shape[1]
    n_pad = nchunks * 1024
    nslabs = nchunks // SLAB

    def count_pred(pred_fn):
        def sbody(s, acc):
            sl = key_ref[0, pl.ds(s * SLAB, SLAB)]
            return acc + jnp.sum(pred_fn(s, sl).astype(jnp.int32))
        return jax.lax.fori_loop(0, nslabs, sbody, jnp.int32(0))

    def count_ge(t):
        return count_pred(lambda s, sl: sl >= t)

    c_pos = count_ge(jnp.int32(0))
    pos_branch = c_pos >= K
    lo0 = jnp.where(pos_branch, jnp.int32(0), jnp.int32(INT_MIN))
    hi0 = jnp.where(pos_branch, jnp.int32(INT_MAX), jnp.int32(-1))

    def bbody(_, lh):
        lo, hi = lh
        d = hi - lo
        mid = lo + (d >> 1) + (d & 1)
        pred = count_ge(mid) >= K
        return (jnp.where(pred, mid, lo), jnp.where(pred, hi, mid - 1))

    t_val, _ = jax.lax.fori_loop(0, 31, bbody, (lo0, hi0))

    c_gt = count_pred(lambda s, sl: sl > t_val)
    c_ge = count_pred(lambda s, sl: sl >= t_val)
    need = K - c_gt

    itie_ref[0] = jnp.int32(n_pad)

    @pl.when(c_ge > K)
    def _tie_search():
        def tbody(_, lh):
            lo, hi = lh
            mid = (lo + hi) >> 1

            def pred_fn(s, sl):
                g = (s * (SLAB * 1024)
                     + jax.lax.broadcasted_iota(jnp.int32, sl.shape, 0) * 1024
                     + jax.lax.broadcasted_iota(jnp.int32, sl.shape, 1) * 128
                     + jax.lax.broadcasted_iota(jnp.int32, sl.shape, 2))
                return (sl == t_val) & (g < mid)

            pred = count_pred(pred_fn) >= need
            return (jnp.where(pred, lo, mid + 1), jnp.where(pred, mid, hi))

        tlo, _ = jax.lax.fori_loop(0, 23, tbody,
                                   (jnp.int32(0), jnp.int32(n_pad)))
        itie_ref[0] = tlo

    i_tie = itie_ref[0]

    liota = (jax.lax.broadcasted_iota(jnp.int32, (8, 128), 0) * 128
             + jax.lax.broadcasted_iota(jnp.int32, (8, 128), 1))

    def chunk_body(r, wpos):
        chunk = key_ref[0, r]
        gidx = r * 1024 + liota
        sel = (chunk > t_val) | ((chunk == t_val) & (gidx < i_tie))
        cnt = jnp.sum(sel.astype(jnp.int32))
        xc = jax.lax.bitcast_convert_type(
            chunk ^ ((chunk >> 31) & jnp.int32(0x7FFFFFFF)), jnp.float32)
        work0 = jnp.where(sel, chunk, jnp.int32(INT_MIN))

        def extract(_, carry):
            work, wp = carry
            mx = jnp.max(work)
            pos = jnp.min(jnp.where(work == mx, liota, jnp.int32(1024)))
            val = jnp.sum(jnp.where(liota == pos, xc, 0.0))
            vals_ref[0, pl.ds(wp, 1), :] = jnp.full((1, 128), val, jnp.float32)
            fidx_ref[0, pl.ds(wp, 1), :] = jnp.full((1, 128), r * 1024 + pos,
                                                    jnp.int32)
            work = jnp.where(liota == pos, jnp.int32(INT_MIN), work)
            return (work, wp + 1)

        _, wpos = jax.lax.fori_loop(0, cnt, extract, (work0, wpos))
        return wpos

    jax.lax.fori_loop(0, nchunks, chunk_body, jnp.int32(0))


def kernel(cls_0, cls_1, cls_2, cls_3, cls_4, box_0, box_1, box_2, box_3, box_4):
    cls_outputs = [cls_0, cls_1, cls_2, cls_3, cls_4]
    box_outputs = [box_0, box_1, box_2, box_3, box_4]
    batch = cls_0.shape[0]
    cls_all = jnp.concatenate(
        [jnp.transpose(c, (0, 2, 3, 1)).reshape(batch, -1, NC) for c in cls_outputs],
        axis=1)
    box_all = jnp.concatenate(
        [jnp.transpose(b, (0, 2, 3, 1)).reshape(batch, -1, 4) for b in box_outputs],
        axis=1)
    n = cls_all.shape[1] * NC
    nchunks = (n + 1023) // 1024
    if nchunks % SLAB:
        nchunks += SLAB - nchunks % SLAB
    n_pad = nchunks * 1024
    cls_flat = cls_all.reshape(batch, n)
    xi = jax.lax.bitcast_convert_type(cls_flat, jnp.int32)
    skey = xi ^ ((xi >> 31) & jnp.int32(0x7FFFFFFF))
    skey = jnp.pad(skey, ((0, 0), (0, n_pad - n)), constant_values=PAD_KEY)
    skey_blocked = skey.reshape(batch, nchunks, 8, 128)

    vals, fidx = pl.pallas_call(
        _select_kernel,
        grid=(batch,),
        in_specs=[pl.BlockSpec((1, nchunks, 8, 128), lambda b: (b, 0, 0, 0))],
        out_specs=[
            pl.BlockSpec((1, K, 128), lambda b: (b, 0, 0)),
            pl.BlockSpec((1, K, 128), lambda b: (b, 0, 0)),
        ],
        out_shape=[
            jax.ShapeDtypeStruct((batch, K, 128), jnp.float32),
            jax.ShapeDtypeStruct((batch, K, 128), jnp.int32),
        ],
        scratch_shapes=[
            pltpu.SMEM((1,), jnp.int32),
        ],
        compiler_params=pltpu.CompilerParams(
            vmem_limit_bytes=60 * 1024 * 1024,
            dimension_semantics=("parallel",),
        ),
    )(skey_blocked)
    vals = vals[:, :, 0]
    fidx = fidx[:, :, 0]

    ord_vals, ord_pos = jax.lax.top_k(vals, K)
    indices_flat = jnp.take_along_axis(fidx, ord_pos, axis=1)
    anchors = indices_flat // NC
    classes = indices_flat % NC
    box_topk = jnp.take_along_axis(box_all, anchors[..., None], axis=1)
    cls_topk = ord_vals[..., None]
    return (cls_topk, box_topk, anchors, classes)
